# baseline (speedup 1.0000x reference)
import functools

import jax
import jax.numpy as jnp
import numpy as np
from jax import lax
from jax.experimental import pallas as pl
from jax.experimental.pallas import tpu as pltpu
from jax.experimental.pallas import tpu_sc as plsc

L = 16
BINS = 256
NSUB = 16
BIG_I = np.int32(2**30)

_mesh = plsc.VectorSubcoreMesh(core_axis_name="c", subcore_axis_name="s",
                               num_cores=1, num_subcores=NSUB)


@functools.partial(
    pl.kernel,
    out_type=jax.ShapeDtypeStruct((L,), jnp.int32),
    mesh=_mesh,
    compiler_params=pltpu.CompilerParams(needs_layout_passes=False,
                                         skip_device_barrier=True),
    scratch_types=[
        pltpu.VMEM((BINS,), jnp.float32),
        pltpu.VMEM((BINS,), jnp.float32),
        pltpu.VMEM((BINS,), jnp.float32),
        pltpu.VMEM((L,), jnp.float32),
        pltpu.VMEM((L,), jnp.int32),
        pltpu.VMEM_SHARED((NSUB * L,), jnp.float32),
        pltpu.VMEM_SHARED((NSUB * L,), jnp.int32),
        pltpu.VMEM((NSUB * L,), jnp.float32),
        pltpu.VMEM((NSUB * L,), jnp.int32),
        pltpu.VMEM((L,), jnp.int32),
    ],
)
def _otsu_sc(hist_hbm, out_hbm, hist_v, p0_v, p1_v, val_v, idx_v,
             sh_val, sh_idx, g_val, g_idx, out_v):
    cid = lax.axis_index("c")
    sid = lax.axis_index("s")

    @pl.when(cid == 0)
    def _body():
        pltpu.sync_copy(hist_hbm, hist_v)

        iota_i = lax.iota(jnp.int32, L)
        iota_f = iota_i.astype(jnp.float32)

        carry0 = jnp.float32(0.0)
        carry1 = jnp.float32(0.0)
        for j in range(BINS // L):
            v = hist_v[pl.ds(L * j, L)]
            w = v * (iota_f + jnp.float32(L * j))
            cs0 = jnp.cumsum(v)
            cs1 = jnp.cumsum(w)
            p0_v[pl.ds(L * j, L)] = cs0 - v + carry0
            p1_v[pl.ds(L * j, L)] = cs1 - w + carry1
            carry0 = carry0 + jnp.sum(v)
            carry1 = carry1 + jnp.sum(w)
        tot0 = carry0
        tot1 = carry1

        def row_body(r, carry):
            bv_r, bi_r = carry
            t1 = 1 + sid + NSUB * r
            t1c = jnp.minimum(t1, BINS - 1)
            baser = jnp.bitwise_and(t1c, np.int32(~(L - 1)))
            lanem = lax.iota(jnp.int32, L) == jnp.bitwise_and(t1c, np.int32(L - 1))
            s0 = jnp.sum(jnp.where(lanem, p0_v[pl.ds(baser, L)], np.float32(0.0)))
            s1 = jnp.sum(jnp.where(lanem, p1_v[pl.ds(baser, L)], np.float32(0.0)))
            m0 = s1 / (s0 + jnp.zeros((L,), jnp.float32))
            cstart = lax.shift_right_logical(t1 + 1, 4)

            def chunk_body(c, carry2):
                bv2, bi2 = carry2
                base = c * L
                p0c = p0_v[pl.ds(base, L)]
                p1c = p1_v[pl.ds(base, L)]
                t2 = base + iota_i
                n1 = p0c - s0
                f1 = p1c - s1
                n2 = tot0 - p0c
                f2 = tot1 - p1c
                m1 = f1 / n1
                m2 = f2 / n2
                d01 = m0 - m1
                d02 = m0 - m2
                d12 = m1 - m2
                var = s0 * n1 * d01 * d01 + s0 * n2 * d02 * d02 \
                    + n1 * n2 * d12 * d12
                var = jnp.where(var != var, jnp.float32(0.0), var)
                valid = (t2 > t1) & (t2 <= BINS - 1)
                var = jnp.where(valid, var, jnp.float32(-1.0))
                idx = t1 * BINS + t2
                upd = (var > bv2) | ((var == bv2) & (idx < bi2))
                return (jnp.where(upd, var, bv2), jnp.where(upd, idx, bi2))

            return lax.fori_loop(cstart, BINS // L, chunk_body, (bv_r, bi_r))

        bv0 = jnp.full((L,), -2.0, jnp.float32)
        bi0 = jnp.full((L,), BIG_I, jnp.int32)
        bv, bi = lax.fori_loop(0, BINS // (L * NSUB) * NSUB, row_body,
                               (bv0, bi0))

        val_v[...] = bv
        idx_v[...] = bi
        pltpu.sync_copy(val_v, sh_val.at[pl.ds(sid * L, L)])
        pltpu.sync_copy(idx_v, sh_idx.at[pl.ds(sid * L, L)])
        plsc.subcore_barrier()

        @pl.when(sid == 0)
        def _finalize():
            pltpu.sync_copy(sh_val, g_val)
            pltpu.sync_copy(sh_idx, g_idx)
            fbv = g_val[pl.ds(0, L)]
            fbi = g_idx[pl.ds(0, L)]
            for srow in range(1, NSUB):
                v2 = g_val[pl.ds(srow * L, L)]
                i2 = g_idx[pl.ds(srow * L, L)]
                upd = (v2 > fbv) | ((v2 == fbv) & (i2 < fbi))
                fbv = jnp.where(upd, v2, fbv)
                fbi = jnp.where(upd, i2, fbi)
            vmax = jnp.max(fbv)
            idxs = jnp.where(fbv == vmax, fbi, BIG_I)
            idx = jnp.min(idxs)
            t1 = lax.shift_right_logical(idx, 8)
            t2 = idx & jnp.int32(BINS - 1)
            outv = jnp.where(iota_i == 0, t1 - 1,
                             jnp.where(iota_i == 1, t2 - 1, jnp.int32(0)))
            out_v[...] = outv
            pltpu.sync_copy(out_v, out_hbm)


def kernel(input, mask, threshold_indices):
    del mask, threshold_indices
    out = _otsu_sc(input)
    return (out[0], out[1])


# speedup vs baseline: 1.0119x; 1.0119x over previous
import functools

import jax
import jax.numpy as jnp
import numpy as np
from jax import lax
from jax.experimental import pallas as pl
from jax.experimental.pallas import tpu as pltpu
from jax.experimental.pallas import tpu_sc as plsc

L = 16
BINS = 256
NSUB = 16
BIG_I = np.int32(2**30)

_mesh = plsc.VectorSubcoreMesh(core_axis_name="c", subcore_axis_name="s",
                               num_cores=1, num_subcores=NSUB)


@functools.partial(
    pl.kernel,
    out_type=jax.ShapeDtypeStruct((L,), jnp.int32),
    mesh=_mesh,
    compiler_params=pltpu.CompilerParams(needs_layout_passes=False,
                                         skip_device_barrier=True),
    scratch_types=[
        pltpu.VMEM((BINS,), jnp.float32),
        pltpu.VMEM((BINS,), jnp.float32),
        pltpu.VMEM((BINS,), jnp.float32),
        pltpu.VMEM((L,), jnp.float32),
        pltpu.VMEM((L,), jnp.int32),
        pltpu.VMEM_SHARED((NSUB * L,), jnp.float32),
        pltpu.VMEM_SHARED((NSUB * L,), jnp.int32),
        pltpu.VMEM((NSUB * L,), jnp.float32),
        pltpu.VMEM((NSUB * L,), jnp.int32),
        pltpu.VMEM((L,), jnp.int32),
    ],
)
def _otsu_sc(hist_hbm, out_hbm, hist_v, p0_v, p1_v, val_v, idx_v,
             sh_val, sh_idx, g_val, g_idx, out_v):
    cid = lax.axis_index("c")
    sid = lax.axis_index("s")

    @pl.when(cid == 0)
    def _body():
        pltpu.sync_copy(hist_hbm, hist_v)

        iota_i = lax.iota(jnp.int32, L)
        iota_f = iota_i.astype(jnp.float32)

        carry0 = jnp.float32(0.0)
        carry1 = jnp.float32(0.0)
        for j in range(BINS // L):
            v = hist_v[pl.ds(L * j, L)]
            w = v * (iota_f + jnp.float32(L * j))
            cs0 = jnp.cumsum(v)
            cs1 = jnp.cumsum(w)
            p0_v[pl.ds(L * j, L)] = cs0 - v + carry0
            p1_v[pl.ds(L * j, L)] = cs1 - w + carry1
            carry0 = carry0 + jnp.sum(v)
            carry1 = carry1 + jnp.sum(w)
        tot0 = carry0
        tot1 = carry1

        def row_body(r, carry):
            bv_r, bi_r = carry
            t1 = 1 + sid + NSUB * r
            t1c = jnp.minimum(t1, BINS - 1)
            baser = jnp.bitwise_and(t1c, np.int32(~(L - 1)))
            lanem = lax.iota(jnp.int32, L) == jnp.bitwise_and(t1c, np.int32(L - 1))
            s0 = jnp.sum(jnp.where(lanem, p0_v[pl.ds(baser, L)], np.float32(0.0)))
            s1 = jnp.sum(jnp.where(lanem, p1_v[pl.ds(baser, L)], np.float32(0.0)))
            cstart = lax.shift_right_logical(t1 + 1, 4)

            def chunk_body(c, carry2):
                bv2, bi2 = carry2
                base = c * L
                p0c = p0_v[pl.ds(base, L)]
                p1c = p1_v[pl.ds(base, L)]
                t2 = base + iota_i
                n1 = p0c - s0
                f1 = p1c - s1
                n2 = tot0 - p0c
                f2 = tot1 - p1c
                a = s1 * n1 - f1 * s0
                b = s1 * n2 - f2 * s0
                cc = f1 * n2 - f2 * n1
                num = n2 * (a * a) + n1 * (b * b) + s0 * (cc * cc)
                den = s0 * n1 * n2
                var = num / den
                bad = (var != var) | (den == np.float32(0.0))
                var = jnp.where(bad, np.float32(0.0), var)
                var = jnp.where(t2 > t1, var, np.float32(-1.0))
                idx = t1 * BINS + t2
                upd = var > bv2
                return (jnp.where(upd, var, bv2), jnp.where(upd, idx, bi2))

            return lax.fori_loop(cstart, BINS // L, chunk_body, (bv_r, bi_r))

        bv0 = jnp.full((L,), -2.0, jnp.float32)
        bi0 = jnp.full((L,), BIG_I, jnp.int32)
        bv, bi = lax.fori_loop(0, BINS // (L * NSUB) * NSUB, row_body,
                               (bv0, bi0))

        val_v[...] = bv
        idx_v[...] = bi
        pltpu.sync_copy(val_v, sh_val.at[pl.ds(sid * L, L)])
        pltpu.sync_copy(idx_v, sh_idx.at[pl.ds(sid * L, L)])
        plsc.subcore_barrier()

        @pl.when(sid == 0)
        def _finalize():
            pltpu.sync_copy(sh_val, g_val)
            pltpu.sync_copy(sh_idx, g_idx)
            fbv = g_val[pl.ds(0, L)]
            fbi = g_idx[pl.ds(0, L)]
            for srow in range(1, NSUB):
                v2 = g_val[pl.ds(srow * L, L)]
                i2 = g_idx[pl.ds(srow * L, L)]
                upd = (v2 > fbv) | ((v2 == fbv) & (i2 < fbi))
                fbv = jnp.where(upd, v2, fbv)
                fbi = jnp.where(upd, i2, fbi)
            vmax = jnp.max(fbv)
            idxs = jnp.where(fbv == vmax, fbi, BIG_I)
            idx = jnp.min(idxs)
            t1 = lax.shift_right_logical(idx, 8)
            t2 = idx & jnp.int32(BINS - 1)
            outv = jnp.where(iota_i == 0, t1 - 1,
                             jnp.where(iota_i == 1, t2 - 1, jnp.int32(0)))
            out_v[...] = outv
            pltpu.sync_copy(out_v, out_hbm)


def kernel(input, mask, threshold_indices):
    del mask, threshold_indices
    out = _otsu_sc(input)
    return (out[0], out[1])
